# Initial kernel scaffold; baseline (speedup 1.0000x reference)
import jax
import jax.numpy as jnp
from jax.experimental import pallas as pl


def kernel(userId, itemId, neg_itemId, user_table, item_table):
    raise NotImplementedError("write your pallas kernel here")


# speedup vs baseline: 1.4252x; 1.4252x over previous
import functools

import jax
import jax.numpy as jnp
from jax import lax
from jax.experimental import pallas as pl
from jax.experimental.pallas import tpu as pltpu
from jax.experimental.pallas import tpu_sc as plsc

B = 16384
D = 128
NC = 2
NS = 16
L = 16
NW = NC * NS
BPW = B // NW
C = 128
NG = BPW // C


def _bpr_body(uid_hbm, pid_hbm, nid_hbm, utab_hbm, itab_hbm,
              pos_hbm, neg_hbm,
              idx_u, idx_p, idx_n, u_rows, p_rows, n_rows,
              pos_v, neg_v, sem_a, sem_b):
    cid = lax.axis_index("c")
    sid = lax.axis_index("s")
    wid = sid * NC + cid

    pltpu.sync_copy(uid_hbm.at[wid], idx_u)
    pltpu.sync_copy(pid_hbm.at[wid], idx_p)
    pltpu.sync_copy(nid_hbm.at[wid], idx_n)

    sems = (sem_a, sem_b)

    def start(g):
        b = g % 2
        return (
            pltpu.async_copy(utab_hbm.at[idx_u.at[g]], u_rows.at[b], sems[b]),
            pltpu.async_copy(itab_hbm.at[idx_p.at[g]], p_rows.at[b], sems[b]),
            pltpu.async_copy(itab_hbm.at[idx_n.at[g]], n_rows.at[b], sems[b]),
        )

    def compute(g):
        b = g % 2
        lane = lax.iota(jnp.int32, L)

        dnums = lax.GatherDimensionNumbers(
            offset_dims=(), collapsed_slice_dims=(0,), start_index_map=(0,))

        def take16(v, idx):
            return lax.gather(
                v, idx[:, None], dnums, slice_sizes=(1,),
                mode=lax.GatherScatterMode.PROMISE_IN_BOUNDS)

        def reduce_all(v):
            for sh in (8, 4, 2, 1):
                v = v + take16(v, lane ^ sh)
            return v

        def grp_body(grp, carry):
            pos_vec = jnp.zeros((L,), jnp.float32)
            neg_vec = jnp.zeros((L,), jnp.float32)
            base_row = grp * L
            for e in range(L):
                row = base_row + e
                accp = accn = None
                for j in range(D // L):
                    u = u_rows[b, row, pl.ds(j * L, L)]
                    p = p_rows[b, row, pl.ds(j * L, L)]
                    nn = n_rows[b, row, pl.ds(j * L, L)]
                    if accp is None:
                        accp, accn = u * p, u * nn
                    else:
                        accp, accn = accp + u * p, accn + u * nn
                pos_vec = jnp.where(lane == e, reduce_all(accp), pos_vec)
                neg_vec = jnp.where(lane == e, reduce_all(accn), neg_vec)
            pos_v[pl.ds(g * C + base_row, L)] = pos_vec
            neg_v[pl.ds(g * C + base_row, L)] = neg_vec
            return carry

        lax.fori_loop(0, C // L, grp_body, 0)

    pending = {0: start(0)}
    for g in range(NG):
        if g + 1 < NG:
            pending[g + 1] = start(g + 1)
        for cp in pending.pop(g):
            cp.wait()
        compute(g)

    pltpu.sync_copy(pos_v, pos_hbm.at[wid])
    pltpu.sync_copy(neg_v, neg_hbm.at[wid])


@jax.jit
def _bpr(uid3, pid3, nid3, user_table, item_table):
    mesh = plsc.VectorSubcoreMesh(core_axis_name="c", subcore_axis_name="s")
    run = functools.partial(
        pl.kernel,
        out_type=(
            jax.ShapeDtypeStruct((NW, BPW), jnp.float32),
            jax.ShapeDtypeStruct((NW, BPW), jnp.float32),
        ),
        mesh=mesh,
        scratch_types=(
            pltpu.VMEM((NG, C), jnp.int32),
            pltpu.VMEM((NG, C), jnp.int32),
            pltpu.VMEM((NG, C), jnp.int32),
            pltpu.VMEM((2, C, D), jnp.float32),
            pltpu.VMEM((2, C, D), jnp.float32),
            pltpu.VMEM((2, C, D), jnp.float32),
            pltpu.VMEM((BPW,), jnp.float32),
            pltpu.VMEM((BPW,), jnp.float32),
            pltpu.SemaphoreType.DMA,
            pltpu.SemaphoreType.DMA,
        ),
    )(_bpr_body)
    return run(uid3, pid3, nid3, user_table, item_table)


def kernel(userId, itemId, neg_itemId, user_table, item_table):
    uid3 = userId.reshape(NW, NG, C)
    pid3 = itemId.reshape(NW, NG, C)
    nid3 = neg_itemId.reshape(NW, NG, C)
    pos, neg = _bpr(uid3, pid3, nid3, user_table, item_table)
    return pos.reshape(B), neg.reshape(B)


# speedup vs baseline: 1.7610x; 1.2356x over previous
import functools

import jax
import jax.numpy as jnp
from jax import lax
from jax.experimental import pallas as pl
from jax.experimental.pallas import tpu as pltpu
from jax.experimental.pallas import tpu_sc as plsc

B = 16384
D = 128
NC = 2
NS = 16
L = 16
NW = NC * NS
BPW = B // NW
C = 128
NG = BPW // C

_BITREV4 = [int(f"{e:04b}"[::-1], 2) for e in range(L)]
_HS = (8, 4, 2, 1)


def _bpr_body(uid_hbm, pid_hbm, nid_hbm, utab_hbm, itab_hbm,
              pos_hbm, neg_hbm,
              idx_u, idx_p, idx_n, u_rows, p_rows, n_rows,
              part_p, part_n, pos_v, neg_v, sem_i, sem_a, sem_b):
    cid = lax.axis_index("c")
    sid = lax.axis_index("s")
    wid = sid * NC + cid

    cps = (pltpu.async_copy(uid_hbm.at[wid], idx_u, sem_i),
           pltpu.async_copy(pid_hbm.at[wid], idx_p, sem_i),
           pltpu.async_copy(nid_hbm.at[wid], idx_n, sem_i))
    for cp in cps:
        cp.wait()

    sems = (sem_a, sem_b)

    def start(g):
        b = g % 2
        return (
            pltpu.async_copy(utab_hbm.at[idx_u.at[g]], u_rows.at[b], sems[b]),
            pltpu.async_copy(itab_hbm.at[idx_p.at[g]], p_rows.at[b], sems[b]),
            pltpu.async_copy(itab_hbm.at[idx_n.at[g]], n_rows.at[b], sems[b]),
        )

    lane = lax.iota(jnp.int32, L)
    dnums = lax.GatherDimensionNumbers(
        offset_dims=(), collapsed_slice_dims=(0,), start_index_map=(0,))

    def take16(v, idx):
        return lax.gather(v, idx[:, None], dnums, slice_sizes=(1,),
                          mode=lax.GatherScatterMode.PROMISE_IN_BOUNDS)

    def combine(a, b, h):
        ta = a + take16(a, lane ^ h)
        tb = b + take16(b, lane ^ h)
        return jnp.where((lane & h) == 0, ta, tb)

    U = 2

    def compute(g):
        b = g % 2

        def rows_body(i, carry):
            for k in range(U):
                row = i * U + k
                u = [u_rows[b, row, pl.ds(j * L, L)] for j in range(D // L)]
                p = [p_rows[b, row, pl.ds(j * L, L)] for j in range(D // L)]
                nn = [n_rows[b, row, pl.ds(j * L, L)] for j in range(D // L)]
                up = [a * c for a, c in zip(u, p)]
                un = [a * c for a, c in zip(u, nn)]
                accp = (((up[0] + up[1]) + (up[2] + up[3]))
                        + ((up[4] + up[5]) + (up[6] + up[7])))
                accn = (((un[0] + un[1]) + (un[2] + un[3]))
                        + ((un[4] + un[5]) + (un[6] + un[7])))
                part_p[pl.ds(row * L, L)] = accp
                part_n[pl.ds(row * L, L)] = accn
            return carry

        lax.fori_loop(0, C // U, rows_body, 0)

        def grp_body(grp, carry):
            base_row = grp * L
            for part, out in ((part_p, pos_v), (part_n, neg_v)):
                stk = []
                for e in _BITREV4:
                    vec = part[pl.ds((base_row + e) * L, L)]
                    lvl = 0
                    while stk and stk[-1][0] == lvl:
                        _, left = stk.pop()
                        vec = combine(left, vec, _HS[lvl])
                        lvl += 1
                    stk.append((lvl, vec))
                out[pl.ds(g * C + base_row, L)] = stk[0][1]
            return carry

        lax.fori_loop(0, C // L, grp_body, 0)

    pending = {0: start(0)}
    for g in range(NG):
        if g + 1 < NG:
            pending[g + 1] = start(g + 1)
        for cp in pending.pop(g):
            cp.wait()
        compute(g)

    base = wid * BPW
    pltpu.sync_copy(pos_v, pos_hbm.at[pl.ds(base, BPW)])
    pltpu.sync_copy(neg_v, neg_hbm.at[pl.ds(base, BPW)])


@jax.jit
def _bpr(uid3, pid3, nid3, user_table, item_table):
    mesh = plsc.VectorSubcoreMesh(core_axis_name="c", subcore_axis_name="s")
    run = functools.partial(
        pl.kernel,
        out_type=(
            jax.ShapeDtypeStruct((B,), jnp.float32),
            jax.ShapeDtypeStruct((B,), jnp.float32),
        ),
        mesh=mesh,
        scratch_types=(
            pltpu.VMEM((NG, C), jnp.int32),
            pltpu.VMEM((NG, C), jnp.int32),
            pltpu.VMEM((NG, C), jnp.int32),
            pltpu.VMEM((2, C, D), jnp.float32),
            pltpu.VMEM((2, C, D), jnp.float32),
            pltpu.VMEM((2, C, D), jnp.float32),
            pltpu.VMEM((C * L,), jnp.float32),
            pltpu.VMEM((C * L,), jnp.float32),
            pltpu.VMEM((BPW,), jnp.float32),
            pltpu.VMEM((BPW,), jnp.float32),
            pltpu.SemaphoreType.DMA,
            pltpu.SemaphoreType.DMA,
            pltpu.SemaphoreType.DMA,
        ),
    )(_bpr_body)
    return run(uid3, pid3, nid3, user_table, item_table)


def kernel(userId, itemId, neg_itemId, user_table, item_table):
    uid3 = userId.reshape(NW, NG, C)
    pid3 = itemId.reshape(NW, NG, C)
    nid3 = neg_itemId.reshape(NW, NG, C)
    return _bpr(uid3, pid3, nid3, user_table, item_table)
